# baseline (speedup 1.0000x reference)
import functools

import jax
import jax.numpy as jnp
from jax import lax
from jax.experimental import pallas as pl
from jax.experimental.pallas import tpu as pltpu
from jax.experimental.pallas import tpu_sc as plsc

R, N = 64, 32768
L = 16
NC, NS = 2, 16
NW = NC * NS
ROWS_PER_W = R // NW
CHUNKS = N // L
SUPV = 8
SUP = SUPV * L
NSUP = N // SUP
ZCH = 4096
BISECT_ITERS = 30


def _row_tau_support(xbuf, cbuf, cidx, cmx, slist):
    lanes = lax.iota(jnp.int32, L)

    @plsc.parallel_loop(0, NSUP, 1, unroll=2,
                        carry=jnp.full((L,), -jnp.inf, jnp.float32))
    def gmaxv(s, acc):
        base = s * SUP
        local = xbuf[pl.ds(base, L)]
        for q in range(1, SUPV):
            local = jnp.maximum(local, xbuf[pl.ds(base + q * L, L)])
        cmx[s] = jnp.max(local)
        return jnp.maximum(acc, local)

    mx = jnp.max(gmaxv)
    e = mx - 1.0

    def sel_body(s, ns):
        slist[ns] = s
        return ns + jnp.where(cmx[s] >= e, 1, 0).astype(jnp.int32)

    ns = lax.fori_loop(0, NSUP, sel_body, jnp.int32(0), unroll=4)

    def cp_sup(k, cnt):
        base = slist[k] * SUP
        c = cnt
        for q in range(SUPV):
            off = base + q * L
            v = xbuf[pl.ds(off, L)]
            m = v >= e
            plsc.store_compressed(cbuf.at[pl.ds(c, L)], v, mask=m)
            plsc.store_compressed(cidx.at[pl.ds(c, L)], off + lanes, mask=m)
            c = c + plsc.all_reduce_population_count(m)[0]
        return c

    cnt = lax.fori_loop(0, ns, cp_sup, jnp.int32(0))
    cbuf[pl.ds(cnt, L)] = jnp.full((L,), -jnp.inf, jnp.float32)
    nch = (cnt + (L - 1)) // L

    def bis_body(_, carry):
        lo, hi = carry
        mid = 0.5 * (lo + hi)

        def g_body(j, acc):
            return acc + jnp.maximum(cbuf[pl.ds(j * L, L)] - mid, 0.0)

        gacc = lax.fori_loop(0, nch, g_body, jnp.zeros((L,), jnp.float32))
        take = jnp.sum(gacc) >= 1.0
        return jnp.where(take, mid, lo), jnp.where(take, hi, mid)

    lo, _hi = lax.fori_loop(0, BISECT_ITERS, bis_body, (e, mx))

    def sc_body(j, carry):
        sacc, cacc = carry
        v = cbuf[pl.ds(j * L, L)]
        m = v > lo
        return (sacc + jnp.where(m, v, 0.0),
                cacc + jnp.where(m, 1.0, 0.0))

    sacc, cacc = lax.fori_loop(
        0, nch, sc_body,
        (jnp.zeros((L,), jnp.float32), jnp.zeros((L,), jnp.float32)))
    num = jnp.broadcast_to(jnp.sum(sacc) - 1.0, (L,))
    den = jnp.broadcast_to(jnp.maximum(jnp.sum(cacc), 1.0), (L,))
    return jnp.max(num / den), cnt


def _sparsemax_body(x_hbm, out_hbm, xbuf, cbuf, cidx, zbuf, cmx, slist,
                    lsem, zsem_a, zsem_b, ssem):
    wid = lax.axis_index("s") * NC + lax.axis_index("c")
    ra = wid * ROWS_PER_W
    lanes = lax.iota(jnp.int32, L)

    load = pltpu.async_copy(x_hbm.at[ra], xbuf, lsem)
    for z in range(ZCH // L):
        zbuf[pl.ds(z * L, L)] = jnp.zeros((L,), jnp.float32)
    for i, zs in enumerate((zsem_a, zsem_b)):
        for k in range(N // ZCH):
            pltpu.async_copy(zbuf, out_hbm.at[ra + i, pl.ds(k * ZCH, ZCH)],
                             zs)

    def process(load, r, zs, next_row):
        load.wait()
        tau, cnt = _row_tau_support(xbuf, cbuf, cidx, cmx, slist)
        padv = jnp.maximum(xbuf[pl.ds(0, L)] - tau, 0.0)

        nload = None
        if next_row is not None:
            nload = pltpu.async_copy(x_hbm.at[next_row], xbuf, lsem)

        nchs = cnt // L + 1

        def val_body(j, carry):
            sl = pl.ds(j * L, L)
            cbuf[sl] = jnp.maximum(cbuf[sl] - tau, 0.0)
            return carry

        lax.fori_loop(0, nchs, val_body, 0)
        cbuf[pl.ds(cnt, L)] = padv
        cidx[pl.ds(cnt, L)] = lanes

        for k in range(N // ZCH):
            pltpu.make_async_copy(
                zbuf, out_hbm.at[r, pl.ds(k * ZCH, ZCH)], zs).wait()

        def scat_body(j, carry):
            sl = pl.ds(j * L, L)
            ireg = cidx[sl]
            pltpu.async_copy(cbuf.at[sl], out_hbm.at[r].at[ireg], ssem)
            return carry

        lax.fori_loop(0, nchs, scat_body, 0)

        def drain_body(j, carry):
            pltpu.make_async_copy(cbuf.at[pl.ds(0, L)],
                                  out_hbm.at[r, pl.ds(0, L)], ssem).wait()
            return carry

        lax.fori_loop(0, nchs, drain_body, 0)
        return nload

    load = process(load, ra, zsem_a, ra + 1)
    process(load, ra + 1, zsem_b, None)


@jax.jit
def kernel(x):
    mesh = plsc.VectorSubcoreMesh(core_axis_name="c", subcore_axis_name="s",
                                  num_cores=NC, num_subcores=NS)
    f = pl.kernel(
        _sparsemax_body,
        out_type=jax.ShapeDtypeStruct((R, N), jnp.float32),
        mesh=mesh,
        scratch_types=[pltpu.VMEM((N,), jnp.float32),
                       pltpu.VMEM((N + L,), jnp.float32),
                       pltpu.VMEM((N + L,), jnp.int32),
                       pltpu.VMEM((ZCH,), jnp.float32),
                       pltpu.SMEM((NSUP,), jnp.float32),
                       pltpu.SMEM((NSUP + 1,), jnp.int32),
                       pltpu.SemaphoreType.DMA,
                       pltpu.SemaphoreType.DMA,
                       pltpu.SemaphoreType.DMA,
                       pltpu.SemaphoreType.DMA],
        compiler_params=pltpu.CompilerParams(needs_layout_passes=False,
                                             use_tc_tiling_on_sc=False),
    )
    return f(x)


# speedup vs baseline: 1.3204x; 1.3204x over previous
import functools

import jax
import jax.numpy as jnp
from jax import lax
from jax.experimental import pallas as pl
from jax.experimental.pallas import tpu as pltpu
from jax.experimental.pallas import tpu_sc as plsc

R, N = 64, 32768
L = 16
NC, NS = 2, 16
NW = NC * NS
ROWS_PER_W = R // NW
CHUNKS = N // L
SUPV = 8
SUP = SUPV * L
NSUP = N // SUP
ZCH = 4096
BISECT_ITERS = 30


def _row_tau(xbuf, cbuf, cmx, slist):
    @plsc.parallel_loop(0, NSUP, 1, unroll=2,
                        carry=jnp.full((L,), -jnp.inf, jnp.float32))
    def gmaxv(s, acc):
        base = s * SUP
        local = xbuf[pl.ds(base, L)]
        for q in range(1, SUPV):
            local = jnp.maximum(local, xbuf[pl.ds(base + q * L, L)])
        cmx[s] = jnp.max(local)
        return jnp.maximum(acc, local)

    mx = jnp.max(gmaxv)
    e = mx - 1.0

    def sel_body(s, ns):
        slist[ns] = s
        return ns + jnp.where(cmx[s] >= e, 1, 0).astype(jnp.int32)

    ns = lax.fori_loop(0, NSUP, sel_body, jnp.int32(0), unroll=4)

    def cp_sup(k, cnt):
        base = slist[k] * SUP
        c = cnt
        for q in range(SUPV):
            v = xbuf[pl.ds(base + q * L, L)]
            m = v >= e
            plsc.store_compressed(cbuf.at[pl.ds(c, L)], v, mask=m)
            c = c + plsc.all_reduce_population_count(m)[0]
        return c

    cnt = lax.fori_loop(0, ns, cp_sup, jnp.int32(0))
    cbuf[pl.ds(cnt, L)] = jnp.full((L,), -jnp.inf, jnp.float32)
    nch = (cnt + (L - 1)) // L

    def bis_body(_, carry):
        lo, hi = carry
        mid = 0.5 * (lo + hi)

        def g_body(j, acc):
            return acc + jnp.maximum(cbuf[pl.ds(j * L, L)] - mid, 0.0)

        gacc = lax.fori_loop(0, nch, g_body, jnp.zeros((L,), jnp.float32))
        take = jnp.sum(gacc) >= 1.0
        return jnp.where(take, mid, lo), jnp.where(take, hi, mid)

    lo, _hi = lax.fori_loop(0, BISECT_ITERS, bis_body, (e, mx))

    def sc_body(j, carry):
        sacc, cacc = carry
        v = cbuf[pl.ds(j * L, L)]
        m = v > lo
        return (sacc + jnp.where(m, v, 0.0),
                cacc + jnp.where(m, 1.0, 0.0))

    sacc, cacc = lax.fori_loop(
        0, nch, sc_body,
        (jnp.zeros((L,), jnp.float32), jnp.zeros((L,), jnp.float32)))
    num = jnp.broadcast_to(jnp.sum(sacc) - 1.0, (L,))
    den = jnp.broadcast_to(jnp.maximum(jnp.sum(cacc), 1.0), (L,))
    return jnp.max(num / den), ns


def _sparsemax_body(x_hbm, out_hbm, xbuf, cbuf, obuf, zbuf, cmx, slist,
                    lsem, zsem_a, zsem_b, ssem):
    wid = lax.axis_index("s") * NC + lax.axis_index("c")
    ra = wid * ROWS_PER_W

    load = pltpu.async_copy(x_hbm.at[ra], xbuf, lsem)
    for z in range(ZCH // L):
        zbuf[pl.ds(z * L, L)] = jnp.zeros((L,), jnp.float32)
    for i, zs in enumerate((zsem_a, zsem_b)):
        for k in range(N // ZCH):
            pltpu.async_copy(zbuf, out_hbm.at[ra + i, pl.ds(k * ZCH, ZCH)],
                             zs)

    def process(load, r, zs, next_row):
        load.wait()
        tau, ns = _row_tau(xbuf, cbuf, cmx, slist)

        def stage_body(k, carry):
            base = slist[k] * SUP
            for q in range(SUPV):
                obuf[pl.ds(k * SUP + q * L, L)] = jnp.maximum(
                    xbuf[pl.ds(base + q * L, L)] - tau, 0.0)
            return carry

        lax.fori_loop(0, ns, stage_body, 0)

        nload = None
        if next_row is not None:
            nload = pltpu.async_copy(x_hbm.at[next_row], xbuf, lsem)

        for k in range(N // ZCH):
            pltpu.make_async_copy(
                zbuf, out_hbm.at[r, pl.ds(k * ZCH, ZCH)], zs).wait()

        def wr_body(k, carry):
            pltpu.async_copy(
                obuf.at[pl.ds(k * SUP, SUP)],
                out_hbm.at[r, pl.ds(slist[k] * SUP, SUP)], ssem)
            return carry

        lax.fori_loop(0, ns, wr_body, 0)

        def drain_body(k, carry):
            pltpu.make_async_copy(obuf.at[pl.ds(0, SUP)],
                                  out_hbm.at[r, pl.ds(0, SUP)], ssem).wait()
            return carry

        lax.fori_loop(0, ns, drain_body, 0)
        return nload

    load = process(load, ra, zsem_a, ra + 1)
    process(load, ra + 1, zsem_b, None)


@jax.jit
def kernel(x):
    mesh = plsc.VectorSubcoreMesh(core_axis_name="c", subcore_axis_name="s",
                                  num_cores=NC, num_subcores=NS)
    f = pl.kernel(
        _sparsemax_body,
        out_type=jax.ShapeDtypeStruct((R, N), jnp.float32),
        mesh=mesh,
        scratch_types=[pltpu.VMEM((N,), jnp.float32),
                       pltpu.VMEM((N + L,), jnp.float32),
                       pltpu.VMEM((N,), jnp.float32),
                       pltpu.VMEM((ZCH,), jnp.float32),
                       pltpu.SMEM((NSUP,), jnp.float32),
                       pltpu.SMEM((NSUP + 1,), jnp.int32),
                       pltpu.SemaphoreType.DMA,
                       pltpu.SemaphoreType.DMA,
                       pltpu.SemaphoreType.DMA,
                       pltpu.SemaphoreType.DMA],
        compiler_params=pltpu.CompilerParams(needs_layout_passes=False),
    )
    return f(x)


# speedup vs baseline: 1.4074x; 1.0659x over previous
import functools

import jax
import jax.numpy as jnp
from jax import lax
from jax.experimental import pallas as pl
from jax.experimental.pallas import tpu as pltpu
from jax.experimental.pallas import tpu_sc as plsc

R, N = 64, 32768
L = 16
NC, NS = 2, 16
NW = NC * NS
ROWS_PER_W = R // NW
CHUNKS = N // L
SUPV = 8
SUP = SUPV * L
NSUP = N // SUP
ZCH = 16384
BISECT_ITERS = 30


def _row_tau(xbuf, cbuf, cmx, slist):
    @plsc.parallel_loop(0, NSUP, 1, unroll=2,
                        carry=jnp.full((L,), -jnp.inf, jnp.float32))
    def gmaxv(s, acc):
        base = s * SUP
        local = xbuf[pl.ds(base, L)]
        for q in range(1, SUPV):
            local = jnp.maximum(local, xbuf[pl.ds(base + q * L, L)])
        cmx[s] = jnp.max(local)
        return jnp.maximum(acc, local)

    mx = jnp.max(gmaxv)
    e = mx - 1.0

    def sel_body(s, ns):
        slist[ns] = s
        return ns + jnp.where(cmx[s] >= e, 1, 0).astype(jnp.int32)

    ns = lax.fori_loop(0, NSUP, sel_body, jnp.int32(0), unroll=4)

    def cp_sup(k, cnt):
        base = slist[k] * SUP
        c = cnt
        for q in range(SUPV):
            v = xbuf[pl.ds(base + q * L, L)]
            m = v >= e
            plsc.store_compressed(cbuf.at[pl.ds(c, L)], v, mask=m)
            c = c + plsc.all_reduce_population_count(m)[0]
        return c

    cnt = lax.fori_loop(0, ns, cp_sup, jnp.int32(0))
    cbuf[pl.ds(cnt, L)] = jnp.full((L,), -jnp.inf, jnp.float32)
    nch = (cnt + (L - 1)) // L

    def bis_body(_, carry):
        lo, hi = carry
        mid = 0.5 * (lo + hi)

        def g_body(j, acc):
            return acc + jnp.maximum(cbuf[pl.ds(j * L, L)] - mid, 0.0)

        gacc = lax.fori_loop(0, nch, g_body, jnp.zeros((L,), jnp.float32))
        take = jnp.sum(gacc) >= 1.0
        return jnp.where(take, mid, lo), jnp.where(take, hi, mid)

    lo, _hi = lax.fori_loop(0, BISECT_ITERS, bis_body, (e, mx))

    def sc_body(j, carry):
        sacc, cacc = carry
        v = cbuf[pl.ds(j * L, L)]
        m = v > lo
        return (sacc + jnp.where(m, v, 0.0),
                cacc + jnp.where(m, 1.0, 0.0))

    sacc, cacc = lax.fori_loop(
        0, nch, sc_body,
        (jnp.zeros((L,), jnp.float32), jnp.zeros((L,), jnp.float32)))
    num = jnp.broadcast_to(jnp.sum(sacc) - 1.0, (L,))
    den = jnp.broadcast_to(jnp.maximum(jnp.sum(cacc), 1.0), (L,))
    return jnp.max(num / den), ns


def _sparsemax_body(x_hbm, out_hbm, bufa, bufb, cbuf, zbuf, cmx, slist,
                    lsa, lsb, zsem_a, zsem_b, ssem):
    wid = lax.axis_index("s") * NC + lax.axis_index("c")
    ra = wid * ROWS_PER_W

    la = pltpu.async_copy(x_hbm.at[ra], bufa, lsa)
    lb = pltpu.async_copy(x_hbm.at[ra + 1], bufb, lsb)
    for z in range(ZCH // L):
        zbuf[pl.ds(z * L, L)] = jnp.zeros((L,), jnp.float32)
    for i, zs in enumerate((zsem_a, zsem_b)):
        for k in range(N // ZCH):
            pltpu.async_copy(zbuf, out_hbm.at[ra + i, pl.ds(k * ZCH, ZCH)],
                             zs)

    def process(load, xbuf, r, zs):
        load.wait()
        tau, ns = _row_tau(xbuf, cbuf, cmx, slist)

        def stage_body(k, carry):
            base = slist[k] * SUP
            for q in range(SUPV):
                sl = pl.ds(base + q * L, L)
                xbuf[sl] = jnp.maximum(xbuf[sl] - tau, 0.0)
            return carry

        lax.fori_loop(0, ns, stage_body, 0)

        for k in range(N // ZCH):
            pltpu.make_async_copy(
                zbuf, out_hbm.at[r, pl.ds(k * ZCH, ZCH)], zs).wait()

        def wr_body(k, carry):
            base = slist[k] * SUP
            pltpu.async_copy(xbuf.at[pl.ds(base, SUP)],
                             out_hbm.at[r, pl.ds(base, SUP)], ssem)
            return carry

        lax.fori_loop(0, ns, wr_body, 0)
        return ns

    nsa = process(la, bufa, ra, zsem_a)
    nsb = process(lb, bufb, ra + 1, zsem_b)

    def drain_body(k, carry):
        pltpu.make_async_copy(bufa.at[pl.ds(0, SUP)],
                              out_hbm.at[ra, pl.ds(0, SUP)], ssem).wait()
        return carry

    lax.fori_loop(0, nsa + nsb, drain_body, 0)


@jax.jit
def kernel(x):
    mesh = plsc.VectorSubcoreMesh(core_axis_name="c", subcore_axis_name="s",
                                  num_cores=NC, num_subcores=NS)
    f = pl.kernel(
        _sparsemax_body,
        out_type=jax.ShapeDtypeStruct((R, N), jnp.float32),
        mesh=mesh,
        scratch_types=[pltpu.VMEM((N,), jnp.float32),
                       pltpu.VMEM((N,), jnp.float32),
                       pltpu.VMEM((N + L,), jnp.float32),
                       pltpu.VMEM((ZCH,), jnp.float32),
                       pltpu.SMEM((NSUP,), jnp.float32),
                       pltpu.SMEM((NSUP + 1,), jnp.int32),
                       pltpu.SemaphoreType.DMA,
                       pltpu.SemaphoreType.DMA,
                       pltpu.SemaphoreType.DMA,
                       pltpu.SemaphoreType.DMA,
                       pltpu.SemaphoreType.DMA],
        compiler_params=pltpu.CompilerParams(needs_layout_passes=False),
    )
    return f(x)


# speedup vs baseline: 1.5136x; 1.0755x over previous
import functools

import jax
import jax.numpy as jnp
from jax import lax
from jax.experimental import pallas as pl
from jax.experimental.pallas import tpu as pltpu
from jax.experimental.pallas import tpu_sc as plsc

R, N = 64, 32768
L = 16
NC, NS = 2, 16
NW = NC * NS
ROWS_PER_W = R // NW
CHUNKS = N // L
SUPV = 8
SUP = SUPV * L
NSUP = N // SUP
BISECT_ITERS = 30
UNROLL = 8


def _row_tau(xbuf, cbuf, cmx, slist, qloads):
    with jax.named_scope("p1_max"):
        acc = jnp.full((L,), -jnp.inf, jnp.float32)
        nq = max(len(qloads), 1)
        for h in range(nq):
            if qloads:
                qloads[h].wait()

            @plsc.parallel_loop(h * NSUP // nq, (h + 1) * NSUP // nq, 1,
                                unroll=2, carry=acc)
            def gmaxv(s, acc):
                base = s * SUP
                local = xbuf[pl.ds(base, L)]
                for q in range(1, SUPV):
                    local = jnp.maximum(local, xbuf[pl.ds(base + q * L, L)])
                cmx[s] = jnp.max(local)
                return jnp.maximum(acc, local)

            acc = gmaxv

    mx = jnp.max(acc)
    e = mx - 1.0

    def sel_body(s, ns):
        slist[ns] = s
        return ns + jnp.where(cmx[s] >= e, 1, 0).astype(jnp.int32)

    with jax.named_scope("p2a_select"):
        ns = lax.fori_loop(0, NSUP, sel_body, jnp.int32(0), unroll=4)

    def cp_sup(k, cnt):
        base = slist[k] * SUP
        c = cnt
        for q in range(SUPV):
            v = xbuf[pl.ds(base + q * L, L)]
            m = v >= e
            plsc.store_compressed(cbuf.at[pl.ds(c, L)], v, mask=m)
            c = c + plsc.all_reduce_population_count(m)[0]
        return c

    with jax.named_scope("p2b_compact"):
        cnt = lax.fori_loop(0, ns, cp_sup, jnp.int32(0))
    cbuf[pl.ds(cnt, L)] = jnp.full((L,), -jnp.inf, jnp.float32)
    nch = (cnt + (L - 1)) // L

    def bis_body(_, carry):
        lo, hi = carry
        mid = 0.5 * (lo + hi)

        def g_body(j, acc):
            return acc + jnp.maximum(cbuf[pl.ds(j * L, L)] - mid, 0.0)

        gacc = lax.fori_loop(0, nch, g_body, jnp.zeros((L,), jnp.float32))
        take = jnp.sum(gacc) >= 1.0
        return jnp.where(take, mid, lo), jnp.where(take, hi, mid)

    with jax.named_scope("p3_bisect"):
        lo, _hi = lax.fori_loop(0, BISECT_ITERS, bis_body, (e, mx))

    def sc_body(j, carry):
        sacc, cacc = carry
        v = cbuf[pl.ds(j * L, L)]
        m = v > lo
        return (sacc + jnp.where(m, v, 0.0),
                cacc + jnp.where(m, 1.0, 0.0))

    sacc, cacc = lax.fori_loop(
        0, nch, sc_body,
        (jnp.zeros((L,), jnp.float32), jnp.zeros((L,), jnp.float32)))
    num = jnp.broadcast_to(jnp.sum(sacc) - 1.0, (L,))
    den = jnp.broadcast_to(jnp.maximum(jnp.sum(cacc), 1.0), (L,))
    return jnp.max(num / den)


def _sparsemax_body(x_hbm, out_hbm, bufa, bufb, cbuf, cmx, slist, lsa, lsb,
                    ssa, ssb):
    wid = lax.axis_index("s") * NC + lax.axis_index("c")
    ra = wid * ROWS_PER_W
    rb = ra + 1

    NQ = 4
    QN = N // NQ
    la = [pltpu.async_copy(x_hbm.at[ra, pl.ds(h * QN, QN)],
                           bufa.at[pl.ds(h * QN, QN)], lsa)
          for h in range(NQ)]
    lb = pltpu.async_copy(x_hbm.at[rb], bufb, lsb)

    def process(qloads, xbuf, r, sem):
        tau = _row_tau(xbuf, cbuf, cmx, slist, qloads)

        outs = []
        with jax.named_scope("p5_out"):
            for h in range(NQ):
                @plsc.parallel_loop(h * QN, (h + 1) * QN, L, unroll=UNROLL)
                def _(j):
                    sl = pl.ds(j, L)
                    xbuf[sl] = jnp.maximum(xbuf[sl] - tau, 0.0)

                outs.append(pltpu.async_copy(
                    xbuf.at[pl.ds(h * QN, QN)],
                    out_hbm.at[r, pl.ds(h * QN, QN)], sem))
        return outs

    sa = process(la, bufa, ra, ssa)
    lb.wait()
    sb = process([], bufb, rb, ssb)
    for s in sa + sb:
        s.wait()


@jax.jit
def kernel(x):
    mesh = plsc.VectorSubcoreMesh(core_axis_name="c", subcore_axis_name="s",
                                  num_cores=NC, num_subcores=NS)
    f = pl.kernel(
        _sparsemax_body,
        out_type=jax.ShapeDtypeStruct((R, N), jnp.float32),
        mesh=mesh,
        scratch_types=[pltpu.VMEM((N,), jnp.float32),
                       pltpu.VMEM((N,), jnp.float32),
                       pltpu.VMEM((N + L,), jnp.float32),
                       pltpu.SMEM((NSUP,), jnp.float32),
                       pltpu.SMEM((NSUP + 1,), jnp.int32),
                       pltpu.SemaphoreType.DMA,
                       pltpu.SemaphoreType.DMA,
                       pltpu.SemaphoreType.DMA,
                       pltpu.SemaphoreType.DMA],
        compiler_params=pltpu.CompilerParams(needs_layout_passes=False),
    )
    return f(x)
